# baseline (speedup 1.0000x reference)
import jax
import jax.numpy as jnp
from jax import lax
from jax.experimental import pallas as pl
from jax.experimental.pallas import tpu as pltpu
from jax.experimental.pallas import tpu_sc as plsc

B = 16384
D = 32
NC = 2
NS = 16
NW = NC * NS
BPW = B // NW
RPB = 128 // D
CHUNK = 128
NCH = BPW // CHUNK
STAGE = 256
NST = BPW // STAGE
LANES = 16


def _cmf_body(users_hbm, items_hbm, uemb_hbm, iemb_hbm, out_hbm,
              uidx_v, iidx_v, ublk_v, iblk_v, ucol_v, icol_v,
              ubuf_v, ibuf_v, out_v, usem, isem):
    wid = lax.axis_index("s") * NC + lax.axis_index("c")
    base = wid * BPW

    pltpu.sync_copy(users_hbm.at[pl.ds(base, BPW)], uidx_v)
    pltpu.sync_copy(items_hbm.at[pl.ds(base, BPW)], iidx_v)

    for t in range(BPW // LANES):
        u = uidx_v[pl.ds(t * LANES, LANES)]
        i = iidx_v[pl.ds(t * LANES, LANES)]
        ublk_v[t // 8, pl.ds((t % 8) * LANES, LANES)] = u >> 2
        iblk_v[t // 8, pl.ds((t % 8) * LANES, LANES)] = i >> 2
        ucol_v[pl.ds(t * LANES, LANES)] = (u & 3) << 5
        icol_v[pl.ds(t * LANES, LANES)] = (i & 3) << 5

    lane = lax.iota(jnp.int32, LANES)

    for s in range(NST):
        copies = []
        for j in range(STAGE // CHUNK):
            jj = s * (STAGE // CHUNK) + j
            copies.append(pltpu.async_copy(
                uemb_hbm.at[ublk_v.at[jj]],
                ubuf_v.at[pl.ds(j * CHUNK, CHUNK)], usem))
            copies.append(pltpu.async_copy(
                iemb_hbm.at[iblk_v.at[jj]],
                ibuf_v.at[pl.ds(j * CHUNK, CHUNK)], isem))
        for c in copies:
            c.wait()

        def chunk16(c, carry):
            g = s * STAGE + c * LANES
            rows = c * LANES + lane
            ucols = ucol_v[pl.ds(g, LANES)]
            icols = icol_v[pl.ds(g, LANES)]
            acc = jnp.zeros((LANES,), jnp.float32)
            for d in range(D):
                ug = plsc.load_gather(ubuf_v, [rows, ucols + d])
                ig = plsc.load_gather(ibuf_v, [rows, icols + d])
                acc = acc + ug * ig
            out_v[pl.ds(g, LANES)] = acc
            return carry

        lax.fori_loop(0, STAGE // LANES, chunk16, 0)

    pltpu.sync_copy(out_v, out_hbm.at[pl.ds(base, BPW)])


@jax.jit
def kernel(users, items, user_emb, item_emb):
    users1 = users.astype(jnp.int32)
    items1 = items.astype(jnp.int32)
    uemb2 = user_emb.reshape(-1, RPB * D)
    iemb2 = item_emb.reshape(-1, RPB * D)
    mesh = plsc.VectorSubcoreMesh(core_axis_name="c", subcore_axis_name="s")
    run = pl.kernel(
        _cmf_body,
        out_type=jax.ShapeDtypeStruct((B,), jnp.float32),
        mesh=mesh,
        compiler_params=pltpu.CompilerParams(needs_layout_passes=False),
        scratch_types=[
            pltpu.VMEM((BPW,), jnp.int32),
            pltpu.VMEM((BPW,), jnp.int32),
            pltpu.VMEM((NCH, CHUNK), jnp.int32),
            pltpu.VMEM((NCH, CHUNK), jnp.int32),
            pltpu.VMEM((BPW,), jnp.int32),
            pltpu.VMEM((BPW,), jnp.int32),
            pltpu.VMEM((STAGE, RPB * D), jnp.float32),
            pltpu.VMEM((STAGE, RPB * D), jnp.float32),
            pltpu.VMEM((BPW,), jnp.float32),
            pltpu.SemaphoreType.DMA,
            pltpu.SemaphoreType.DMA,
        ],
    )
    return run(users1, items1, uemb2, iemb2)


# speedup vs baseline: 3.7382x; 3.7382x over previous
import jax
import jax.numpy as jnp
from jax import lax
from jax.experimental import pallas as pl
from jax.experimental.pallas import tpu as pltpu
from jax.experimental.pallas import tpu_sc as plsc

B = 16384
D = 32
N = 1000000
NC = 2
NS = 16
NW = NC * NS
BPW = B // NW
GRP = 16
NG = BPW // GRP
LANES = 16
W = 128


def _cmf_body(users_hbm, items_hbm, uembt_hbm, iembt_hbm, out_hbm,
              uidx_v, iidx_v, win_v, ucol_v, out_v, sem):
    wid = lax.axis_index("s") * NC + lax.axis_index("c")
    base = wid * BPW

    pltpu.sync_copy(users_hbm.at[pl.ds(base, BPW)], uidx_v)
    pltpu.sync_copy(items_hbm.at[pl.ds(base, BPW)], iidx_v)

    lane = lax.iota(jnp.int32, LANES)

    def group(g, carry):
        gbase = g * GRP

        uvec = uidx_v[pl.ds(gbase, GRP)]
        copies = []
        for e in range(GRP):
            off = pl.multiple_of((uvec[e] >> 7) * W, W)
            copies.append(pltpu.async_copy(
                uembt_hbm.at[:, pl.ds(off, W)],
                win_v.at[pl.ds(e * D, D)], sem))
        for cp in copies:
            cp.wait()

        ul = uvec & 127
        for d in range(D):
            ucol_v[d] = plsc.load_gather(win_v, [lane * D + d, ul])

        ivec = iidx_v[pl.ds(gbase, GRP)]
        copies = []
        for e in range(GRP):
            off = pl.multiple_of((ivec[e] >> 7) * W, W)
            copies.append(pltpu.async_copy(
                iembt_hbm.at[:, pl.ds(off, W)],
                win_v.at[pl.ds(e * D, D)], sem))
        for cp in copies:
            cp.wait()

        il = ivec & 127
        acc = ucol_v[0] * plsc.load_gather(win_v, [lane * D, il])
        for d in range(1, D):
            acc = acc + ucol_v[d] * plsc.load_gather(
                win_v, [lane * D + d, il])
        out_v[pl.ds(gbase, GRP)] = acc
        return carry

    lax.fori_loop(0, NG, group, 0)

    pltpu.sync_copy(out_v, out_hbm.at[pl.ds(base, BPW)])


@jax.jit
def kernel(users, items, user_emb, item_emb):
    users1 = users.astype(jnp.int32)
    items1 = items.astype(jnp.int32)
    uembt = user_emb.T
    iembt = item_emb.T
    mesh = plsc.VectorSubcoreMesh(core_axis_name="c", subcore_axis_name="s")
    run = pl.kernel(
        _cmf_body,
        out_type=jax.ShapeDtypeStruct((B,), jnp.float32),
        mesh=mesh,
        compiler_params=pltpu.CompilerParams(needs_layout_passes=False),
        scratch_types=[
            pltpu.VMEM((BPW,), jnp.int32),
            pltpu.VMEM((BPW,), jnp.int32),
            pltpu.VMEM((GRP * D, W), jnp.float32),
            pltpu.VMEM((D, GRP), jnp.float32),
            pltpu.VMEM((BPW,), jnp.float32),
            pltpu.SemaphoreType.DMA,
        ],
    )
    return run(users1, items1, uembt, iembt)
